# baseline (speedup 1.0000x reference)
import functools

import jax
import jax.numpy as jnp
from jax import lax
from jax.experimental import pallas as pl
from jax.experimental.pallas import tpu as pltpu
from jax.experimental.pallas import tpu_sc as plsc

D = 64
DW = 128
B = 1024
S = 200
L = S + 1

NC = 2
NS = 16
NW = NC * NS
BPW = B // NW
CHUNK_A = 104
CHUNK_B = 96
ROWW = L * DW
SQRT_D = 8.0


def _pos_embedding():
    num_ts = D // 2
    position = jnp.arange(S, dtype=jnp.float32)[:, None]
    log_inc = jnp.log(jnp.float32(10000.0)) / jnp.maximum(
        jnp.float32(num_ts) - 1.0, 1.0)
    inv_ts = jnp.exp(jnp.arange(num_ts, dtype=jnp.float32) * -log_inc)
    scaled = position * inv_ts[None, :]
    return jnp.concatenate([jnp.sin(scaled), jnp.cos(scaled)], axis=-1)


@functools.partial(
    pl.kernel,
    out_type=jax.ShapeDtypeStruct((B * ROWW,), jnp.float32),
    mesh=plsc.VectorSubcoreMesh(core_axis_name="c", subcore_axis_name="s"),
    scratch_types=[
        pltpu.VMEM((CHUNK_A,), jnp.int32),
        pltpu.VMEM((CHUNK_B,), jnp.int32),
        pltpu.VMEM((S, DW), jnp.float32),
        pltpu.VMEM((ROWW,), jnp.float32),
        pltpu.VMEM((S * D,), jnp.float32),
        pltpu.SemaphoreType.DMA,
    ],
    compiler_params=pltpu.CompilerParams(use_tc_tiling_on_sc=False),
)
def _emb_sc(ids_hbm, table_hbm, pos_hbm, cls_hbm, x_hbm,
            idxa_v, idxb_v, raw_v, out_v, pos_v, sem):
    wid = lax.axis_index("s") * NC + lax.axis_index("c")
    pltpu.sync_copy(pos_hbm, pos_v)
    pltpu.sync_copy(cls_hbm, out_v.at[pl.ds(S * DW, D)])

    def batch_body(i, _):
        bb = wid * BPW + i
        pltpu.sync_copy(ids_hbm.at[bb, pl.ds(0, CHUNK_A)], idxa_v)
        pltpu.sync_copy(ids_hbm.at[bb, pl.ds(CHUNK_A, CHUNK_B)], idxb_v)
        c0 = pltpu.async_copy(table_hbm.at[idxa_v],
                              raw_v.at[pl.ds(0, CHUNK_A)], sem)
        c1 = pltpu.async_copy(table_hbm.at[idxb_v],
                              raw_v.at[pl.ds(CHUNK_A, CHUNK_B)], sem)
        c0.wait()
        c1.wait()

        @plsc.parallel_loop(0, S, unroll=8)
        def row_body(r):
            for c in range(D // 16):
                sl16 = pl.ds(c * 16, 16)
                out_v[pl.ds(r * DW + c * 16, 16)] = (
                    raw_v[r, sl16] * SQRT_D + pos_v[pl.ds(r * D + c * 16, 16)])

        pltpu.sync_copy(out_v, x_hbm.at[pl.ds(bb * ROWW, ROWW)])
        return 0

    lax.fori_loop(0, BPW, batch_body, 0)


_MASK_RI = 3


def _mask_body(padT_ref, o_ref):
    p = pl.program_id(0)
    padT = padT_ref[...]
    jio = lax.broadcasted_iota(jnp.int32, (L, B), 0)
    zero = jnp.zeros((L, B), jnp.int32)
    for k in range(_MASK_RI):
        i = p * _MASK_RI + k
        o_ref[k] = jnp.where(jio <= i, padT, zero)


_mask_call = pl.pallas_call(
    _mask_body,
    grid=(L // _MASK_RI,),
    in_specs=[pl.BlockSpec((L, B), lambda i: (0, 0))],
    out_specs=pl.BlockSpec((_MASK_RI, L, B), lambda i: (i, 0, 0)),
    out_shape=jax.ShapeDtypeStruct((L, L, B), jnp.int32),
)


def kernel(token_ids, padding_mask, table, cls_token):
    ids = token_ids.astype(jnp.int32)
    table_w = jnp.pad(table, ((0, 0), (0, DW - D)))
    pos = _pos_embedding().reshape(S * D)
    cls8 = (cls_token * SQRT_D).reshape(D)
    x_flat = _emb_sc(ids, table_w, pos, cls8)
    x = x_flat.reshape(B, L, DW)[:, :, :D]
    padT_full = jnp.concatenate(
        [padding_mask.astype(jnp.int32).T, jnp.ones((1, B), jnp.int32)], axis=0)
    maskT = _mask_call(padT_full)
    mask = jnp.transpose(maskT, (2, 0, 1))
    return x, mask


# speedup vs baseline: 1.0526x; 1.0526x over previous
import functools

import jax
import jax.numpy as jnp
from jax import lax
from jax.experimental import pallas as pl
from jax.experimental.pallas import tpu as pltpu
from jax.experimental.pallas import tpu_sc as plsc

D = 64
DW = 128
B = 1024
S = 200
L = S + 1

NC = 2
NS = 16
NW = NC * NS
BPW = B // NW
CHUNK_A = 104
CHUNK_B = 96
SQRT_D = 8.0


def _pos_embedding():
    num_ts = D // 2
    position = jnp.arange(S, dtype=jnp.float32)[:, None]
    log_inc = jnp.log(jnp.float32(10000.0)) / jnp.maximum(
        jnp.float32(num_ts) - 1.0, 1.0)
    inv_ts = jnp.exp(jnp.arange(num_ts, dtype=jnp.float32) * -log_inc)
    scaled = position * inv_ts[None, :]
    return jnp.concatenate([jnp.sin(scaled), jnp.cos(scaled)], axis=-1)


@functools.partial(
    pl.kernel,
    out_type=jax.ShapeDtypeStruct((B, L, DW), jnp.float32),
    mesh=plsc.VectorSubcoreMesh(core_axis_name="c", subcore_axis_name="s"),
    scratch_types=[
        pltpu.VMEM((CHUNK_A,), jnp.int32),
        pltpu.VMEM((CHUNK_B,), jnp.int32),
        pltpu.VMEM((CHUNK_A,), jnp.int32),
        pltpu.VMEM((CHUNK_B,), jnp.int32),
        pltpu.VMEM((S, DW), jnp.float32),
        pltpu.VMEM((S, DW), jnp.float32),
        pltpu.VMEM((L, DW), jnp.float32),
        pltpu.VMEM((S * D,), jnp.float32),
        pltpu.SemaphoreType.DMA,
        pltpu.SemaphoreType.DMA,
    ],
    compiler_params=pltpu.CompilerParams(use_tc_tiling_on_sc=False),
)
def _emb_sc(ids_hbm, table_hbm, pos_hbm, cls_hbm, x_hbm,
            idxa0, idxb0, idxa1, idxb1, raw0, raw1, out_v, pos_v,
            sem0, sem1):
    wid = lax.axis_index("s") * NC + lax.axis_index("c")
    pltpu.sync_copy(pos_hbm, pos_v)
    pltpu.sync_copy(cls_hbm, out_v.at[pl.ds(S, 1)])
    b0 = wid * BPW

    def issue(bb, idxa, idxb, raw, sem):
        pltpu.sync_copy(ids_hbm.at[bb, pl.ds(0, CHUNK_A)], idxa)
        pltpu.sync_copy(ids_hbm.at[bb, pl.ds(CHUNK_A, CHUNK_B)], idxb)
        c0 = pltpu.async_copy(table_hbm.at[idxa],
                              raw.at[pl.ds(0, CHUNK_A)], sem)
        c1 = pltpu.async_copy(table_hbm.at[idxb],
                              raw.at[pl.ds(CHUNK_A, CHUNK_B)], sem)
        return c0, c1

    def drain(bb, raw, sem):
        pltpu.make_async_copy(table_hbm.at[idxa0 if raw is raw0 else idxa1],
                              raw.at[pl.ds(0, CHUNK_A)], sem).wait()
        pltpu.make_async_copy(table_hbm.at[idxb0 if raw is raw0 else idxb1],
                              raw.at[pl.ds(CHUNK_A, CHUNK_B)], sem).wait()

    def epilogue(bb, raw):
        @plsc.parallel_loop(0, S, unroll=8)
        def row_body(r):
            for c in range(D // 16):
                sl16 = pl.ds(c * 16, 16)
                out_v[r, sl16] = (
                    raw[r, sl16] * SQRT_D + pos_v[pl.ds(r * D + c * 16, 16)])

        pltpu.sync_copy(out_v, x_hbm.at[bb])

    issue(b0, idxa0, idxb0, raw0, sem0)

    def pair_body(k, _):
        even = b0 + 2 * k
        drain(even, raw0, sem0)
        issue(even + 1, idxa1, idxb1, raw1, sem1)
        epilogue(even, raw0)
        drain(even + 1, raw1, sem1)

        @pl.when(k < BPW // 2 - 1)
        def _():
            issue(even + 2, idxa0, idxb0, raw0, sem0)

        epilogue(even + 1, raw1)
        return 0

    lax.fori_loop(0, BPW // 2, pair_body, 0)


_MASK_RI = 3


def _mask_body(padT_ref, o_ref):
    p = pl.program_id(0)
    padT = padT_ref[...]
    jio = lax.broadcasted_iota(jnp.int32, (L, B), 0)
    zero = jnp.zeros((L, B), jnp.int32)
    for k in range(_MASK_RI):
        i = p * _MASK_RI + k
        o_ref[k] = jnp.where(jio <= i, padT, zero)


_mask_call = pl.pallas_call(
    _mask_body,
    grid=(L // _MASK_RI,),
    in_specs=[pl.BlockSpec((L, B), lambda i: (0, 0))],
    out_specs=pl.BlockSpec((_MASK_RI, L, B), lambda i: (i, 0, 0)),
    out_shape=jax.ShapeDtypeStruct((L, L, B), jnp.int32),
)


def kernel(token_ids, padding_mask, table, cls_token):
    padT_full = jnp.concatenate(
        [padding_mask.astype(jnp.int32).T, jnp.ones((1, B), jnp.int32)], axis=0)
    maskT = _mask_call(padT_full)
    mask = jnp.transpose(maskT, (2, 0, 1))
    ids = token_ids.astype(jnp.int32)
    table_w = jnp.pad(table, ((0, 0), (0, DW - D)))
    pos = _pos_embedding().reshape(S * D)
    cls8 = jnp.pad(cls_token.reshape(1, D) * SQRT_D, ((0, 0), (0, DW - D)))
    x_wide = _emb_sc(ids, table_w, pos, cls8)
    x = x_wide[:, :, :D]
    return x, mask


# speedup vs baseline: 1.1782x; 1.1193x over previous
import functools

import jax
import jax.numpy as jnp
from jax import lax
from jax.experimental import pallas as pl
from jax.experimental.pallas import tpu as pltpu
from jax.experimental.pallas import tpu_sc as plsc

D = 64
DW = 128
B = 1024
S = 200
SW = 256
L = S + 1
LW = 208

NC = 2
NS = 16
NW = NC * NS
BPW = B // NW
SQRT_D = 8.0


def _pos_embedding():
    num_ts = D // 2
    position = jnp.arange(S, dtype=jnp.float32)[:, None]
    log_inc = jnp.log(jnp.float32(10000.0)) / jnp.maximum(
        jnp.float32(num_ts) - 1.0, 1.0)
    inv_ts = jnp.exp(jnp.arange(num_ts, dtype=jnp.float32) * -log_inc)
    scaled = position * inv_ts[None, :]
    return jnp.concatenate([jnp.sin(scaled), jnp.cos(scaled)], axis=-1)


@functools.partial(
    pl.kernel,
    out_type=jax.ShapeDtypeStruct((B, LW, DW), jnp.float32),
    mesh=plsc.VectorSubcoreMesh(core_axis_name="c", subcore_axis_name="s"),
    scratch_types=[
        pltpu.VMEM((BPW, DW), jnp.int32),
        pltpu.VMEM((BPW, DW), jnp.int32),
        pltpu.VMEM((SW, DW), jnp.float32),
        pltpu.VMEM((SW, DW), jnp.float32),
        pltpu.VMEM((LW, DW), jnp.float32),
        pltpu.VMEM((S * D,), jnp.float32),
        pltpu.VMEM((DW,), jnp.float32),
        pltpu.SemaphoreType.DMA,
        pltpu.SemaphoreType.DMA,
    ],
    compiler_params=pltpu.CompilerParams(use_tc_tiling_on_sc=True),
)
def _emb_sc(ids_hbm, table_hbm, pos_hbm, cls_hbm, x_hbm,
            idsa_v, idsb_v, raw0, raw1, out_v, pos_v, cls_v, sem0, sem1):
    wid = lax.axis_index("s") * NC + lax.axis_index("c")
    b0 = wid * BPW
    pltpu.sync_copy(pos_hbm, pos_v)
    pltpu.sync_copy(cls_hbm, cls_v)
    pltpu.sync_copy(ids_hbm.at[pl.ds(b0, BPW), pl.ds(0, DW)], idsa_v)
    pltpu.sync_copy(ids_hbm.at[pl.ds(b0, BPW), pl.ds(DW, DW)], idsb_v)
    for c in range(D // 16):
        sl16 = pl.ds(c * 16, 16)
        out_v[S, sl16] = cls_v[sl16]

    def issue(i, raw, sem):
        c0 = pltpu.async_copy(table_hbm.at[idsa_v.at[i]],
                              raw.at[pl.ds(0, DW)], sem)
        c1 = pltpu.async_copy(table_hbm.at[idsb_v.at[i]],
                              raw.at[pl.ds(DW, DW)], sem)
        return c0, c1

    def drain(i, raw, sem):
        pltpu.make_async_copy(table_hbm.at[idsa_v.at[i]],
                              raw.at[pl.ds(0, DW)], sem).wait()
        pltpu.make_async_copy(table_hbm.at[idsb_v.at[i]],
                              raw.at[pl.ds(DW, DW)], sem).wait()

    def epilogue(i, raw):
        @plsc.parallel_loop(0, S, unroll=8)
        def row_body(r):
            for c in range(D // 16):
                sl16 = pl.ds(c * 16, 16)
                out_v[r, sl16] = (
                    raw[r, sl16] * SQRT_D + pos_v[pl.ds(r * D + c * 16, 16)])

        pltpu.sync_copy(out_v, x_hbm.at[b0 + i])

    issue(0, raw0, sem0)

    def pair_body(k, _):
        even = 2 * k
        drain(even, raw0, sem0)
        issue(even + 1, raw1, sem1)
        epilogue(even, raw0)
        drain(even + 1, raw1, sem1)

        @pl.when(k < BPW // 2 - 1)
        def _():
            issue(even + 2, raw0, sem0)

        epilogue(even + 1, raw1)
        return 0

    lax.fori_loop(0, BPW // 2, pair_body, 0)


_MASK_RI = 3


def _mask_body(padT_ref, o_ref):
    p = pl.program_id(0)
    padT = padT_ref[...]
    jio = lax.broadcasted_iota(jnp.int32, (L, B), 0)
    zero = jnp.zeros((L, B), jnp.int32)
    for k in range(_MASK_RI):
        i = p * _MASK_RI + k
        o_ref[k] = jnp.where(jio <= i, padT, zero)


_mask_call = pl.pallas_call(
    _mask_body,
    grid=(L // _MASK_RI,),
    in_specs=[pl.BlockSpec((L, B), lambda i: (0, 0))],
    out_specs=pl.BlockSpec((_MASK_RI, L, B), lambda i: (i, 0, 0)),
    out_shape=jax.ShapeDtypeStruct((L, L, B), jnp.int32),
)


def kernel(token_ids, padding_mask, table, cls_token):
    padT_full = jnp.concatenate(
        [padding_mask.astype(jnp.int32).T, jnp.ones((1, B), jnp.int32)], axis=0)
    maskT = _mask_call(padT_full)
    mask = jnp.transpose(maskT, (2, 0, 1))
    ids = token_ids.astype(jnp.int32)
    ids_w = jnp.concatenate([ids, ids[:, : SW - S]], axis=1)
    table_w = jnp.pad(table, ((0, 0), (0, DW - D)))
    pos = _pos_embedding().reshape(S * D)
    cls128 = jnp.pad(cls_token.reshape(1, D) * SQRT_D,
                     ((0, 0), (0, DW - D))).reshape(DW)
    x_wide = _emb_sc(ids_w, table_w, pos, cls128)
    x = x_wide[:, :L, :D]
    return x, mask


# speedup vs baseline: 1.2004x; 1.0188x over previous
import functools

import jax
import jax.numpy as jnp
from jax import lax
from jax.experimental import pallas as pl
from jax.experimental.pallas import tpu as pltpu
from jax.experimental.pallas import tpu_sc as plsc

D = 64
DW = 128
B = 1024
S = 200
SB = 72
L = S + 1
LW = 208

NC = 2
NS = 16
NW = NC * NS
BPW = B // NW
SQRT_D = 8.0


def _pos_embedding():
    num_ts = D // 2
    position = jnp.arange(S, dtype=jnp.float32)[:, None]
    log_inc = jnp.log(jnp.float32(10000.0)) / jnp.maximum(
        jnp.float32(num_ts) - 1.0, 1.0)
    inv_ts = jnp.exp(jnp.arange(num_ts, dtype=jnp.float32) * -log_inc)
    scaled = position * inv_ts[None, :]
    return jnp.concatenate([jnp.sin(scaled), jnp.cos(scaled)], axis=-1)


@functools.partial(
    pl.kernel,
    out_type=jax.ShapeDtypeStruct((B, LW, DW), jnp.float32),
    mesh=plsc.VectorSubcoreMesh(core_axis_name="c", subcore_axis_name="s"),
    scratch_types=[
        pltpu.VMEM((BPW, DW), jnp.int32),
        pltpu.VMEM((BPW, SB), jnp.int32),
        pltpu.VMEM((S, DW), jnp.float32),
        pltpu.VMEM((S, DW), jnp.float32),
        pltpu.VMEM((LW, DW), jnp.float32),
        pltpu.VMEM((S * D,), jnp.float32),
        pltpu.VMEM((DW,), jnp.float32),
        pltpu.SemaphoreType.DMA,
        pltpu.SemaphoreType.DMA,
    ],
    compiler_params=pltpu.CompilerParams(use_tc_tiling_on_sc=True),
)
def _emb_sc(ids_hbm, table_hbm, pos_hbm, cls_hbm, x_hbm,
            idsa_v, idsb_v, raw0, raw1, out_v, pos_v, cls_v, sem0, sem1):
    wid = lax.axis_index("s") * NC + lax.axis_index("c")
    b0 = wid * BPW
    pltpu.sync_copy(pos_hbm, pos_v)
    pltpu.sync_copy(cls_hbm, cls_v)
    pltpu.sync_copy(ids_hbm.at[pl.ds(b0, BPW), pl.ds(0, DW)], idsa_v)
    pltpu.sync_copy(ids_hbm.at[pl.ds(b0, BPW), pl.ds(DW, SB)], idsb_v)
    for c in range(D // 16):
        sl16 = pl.ds(c * 16, 16)
        out_v[S, sl16] = cls_v[sl16]

    def issue(i, raw, sem):
        c0 = pltpu.async_copy(table_hbm.at[idsa_v.at[i]],
                              raw.at[pl.ds(0, DW)], sem)
        c1 = pltpu.async_copy(table_hbm.at[idsb_v.at[i]],
                              raw.at[pl.ds(DW, SB)], sem)
        return c0, c1

    def drain(i, raw, sem):
        pltpu.make_async_copy(table_hbm.at[idsa_v.at[i]],
                              raw.at[pl.ds(0, DW)], sem).wait()
        pltpu.make_async_copy(table_hbm.at[idsb_v.at[i]],
                              raw.at[pl.ds(DW, SB)], sem).wait()

    def epilogue(i, raw):
        @plsc.parallel_loop(0, S, unroll=8)
        def row_body(r):
            for c in range(D // 16):
                sl16 = pl.ds(c * 16, 16)
                out_v[r, sl16] = (
                    raw[r, sl16] * SQRT_D + pos_v[pl.ds(r * D + c * 16, 16)])

        pltpu.sync_copy(out_v, x_hbm.at[b0 + i])

    issue(0, raw0, sem0)

    def pair_body(k, _):
        even = 2 * k
        drain(even, raw0, sem0)
        issue(even + 1, raw1, sem1)
        epilogue(even, raw0)
        drain(even + 1, raw1, sem1)

        @pl.when(k < BPW // 2 - 1)
        def _():
            issue(even + 2, raw0, sem0)

        epilogue(even + 1, raw1)
        return 0

    lax.fori_loop(0, BPW // 2, pair_body, 0)


_MASK_RI = 3


def _mask_body(padT_ref, o_ref):
    p = pl.program_id(0)
    padT = padT_ref[...]
    jio = lax.broadcasted_iota(jnp.int32, (L, B), 0)
    zero = jnp.zeros((L, B), jnp.int32)
    for k in range(_MASK_RI):
        i = p * _MASK_RI + k
        o_ref[k] = jnp.where(jio <= i, padT, zero)


_mask_call = pl.pallas_call(
    _mask_body,
    grid=(L // _MASK_RI,),
    in_specs=[pl.BlockSpec((L, B), lambda i: (0, 0))],
    out_specs=pl.BlockSpec((_MASK_RI, L, B), lambda i: (i, 0, 0)),
    out_shape=jax.ShapeDtypeStruct((L, L, B), jnp.int32),
)


def kernel(token_ids, padding_mask, table, cls_token):
    padT_full = jnp.concatenate(
        [padding_mask.astype(jnp.int32).T, jnp.ones((1, B), jnp.int32)], axis=0)
    maskT = _mask_call(padT_full)
    mask = jnp.transpose(maskT, (2, 0, 1))
    ids = token_ids.astype(jnp.int32)
    table_w = jnp.pad(table, ((0, 0), (0, DW - D)))
    pos = _pos_embedding().reshape(S * D)
    cls128 = jnp.pad(cls_token.reshape(1, D) * SQRT_D,
                     ((0, 0), (0, DW - D))).reshape(DW)
    x_wide = _emb_sc(ids, table_w, pos, cls128)
    x = x_wide[:, :L, :D]
    return x, mask
